# baseline (speedup 1.0000x reference)
import jax
import jax.numpy as jnp
from jax import lax
from jax.experimental import pallas as pl
from jax.experimental.pallas import tpu as pltpu

TOP_K = 64
NHIDDEN = 16
D = 512
N = 1194
GROUPS = 6
GLEN = 199
BLK_M = 256
GRID_M = 5


def _body(d1_ref, d2_ref, W_ref, W1_ref, b1_ref, W2_ref, b2_ref, out_ref,
          s2_ref):
    f32 = jnp.float32
    i = pl.program_id(0)

    diff = d1_ref[...] - d2_ref[...]
    E = jnp.dot(diff, W_ref[...], preferred_element_type=f32)
    Ee = E + 1e-6
    s2_blk = jnp.dot(Ee * Ee, jnp.ones((D, 1), f32),
                     preferred_element_type=f32)
    s2_ref[pl.ds(i * BLK_M, BLK_M), :] = s2_blk

    @pl.when(i == GRID_M - 1)
    def _epilogue():
        eye = (lax.broadcasted_iota(jnp.int32, (GLEN, GLEN), 0)
               == lax.broadcasted_iota(jnp.int32, (GLEN, GLEN), 1)).astype(f32)
        subio = lax.broadcasted_iota(jnp.int32, (GLEN, GLEN), 0)
        lanio = lax.broadcasted_iota(jnp.int32, (GLEN, GLEN), 1)
        ones_col = jnp.ones((GLEN, 1), f32)
        kiof = lax.broadcasted_iota(jnp.int32, (GLEN, TOP_K), 1).astype(f32)

        cols = [s2_ref[pl.ds(g * GLEN, GLEN), :] for g in range(GROUPS)]
        colall = jnp.concatenate(cols, axis=1)
        rowall = lax.dot_general(
            colall, eye, dimension_numbers=(((0,), (0,)), ((), ())),
            preferred_element_type=f32)

        xs_rows = []
        for g in range(GROUPS):
            rowg = rowall[g:g + 1, :]
            colg = colall[:, g:g + 1]
            gt = rowg > colg
            tie = (rowg == colg) & (lanio < subio)
            cnt = gt.astype(f32) + tie.astype(f32)
            rank = lax.dot_general(
                cnt, ones_col, dimension_numbers=(((1,), (0,)), ((), ())),
                preferred_element_type=f32)
            oh = (rank == kiof).astype(f32)
            xs_rows.append(lax.dot_general(
                rowg, oh, dimension_numbers=(((1,), (0,)), ((), ())),
                preferred_element_type=f32))

        xs = jnp.concatenate(xs_rows, axis=0)
        x = jnp.sqrt(xs)
        h = jnp.maximum(
            jnp.dot(x, W1_ref[...], preferred_element_type=f32)
            + b1_ref[...], 0.0)
        out_ref[...] = (
            jnp.dot(h, W2_ref[...], preferred_element_type=f32)
            + b2_ref[...])


def kernel(data1, data2, W_emb, b_emb, W1, b1, W2, b2):
    del b_emb
    out = pl.pallas_call(
        _body,
        grid=(GRID_M,),
        in_specs=[
            pl.BlockSpec((BLK_M, D), lambda i: (i, 0)),
            pl.BlockSpec((BLK_M, D), lambda i: (i, 0)),
            pl.BlockSpec((D, D), lambda i: (0, 0)),
            pl.BlockSpec((TOP_K, NHIDDEN), lambda i: (0, 0)),
            pl.BlockSpec((1, NHIDDEN), lambda i: (0, 0)),
            pl.BlockSpec((NHIDDEN, 1), lambda i: (0, 0)),
            pl.BlockSpec((1, 1), lambda i: (0, 0)),
        ],
        out_specs=pl.BlockSpec((GROUPS, 1), lambda i: (0, 0)),
        out_shape=jax.ShapeDtypeStruct((GROUPS, 1), jnp.float32),
        scratch_shapes=[pltpu.VMEM((BLK_M * GRID_M, 1), jnp.float32)],
    )(data1, data2, W_emb, W1, b1.reshape(1, NHIDDEN), W2, b2.reshape(1, 1))
    return out
